# baseline (speedup 1.0000x reference)
import functools

import jax
import jax.numpy as jnp
from jax import lax
from jax.experimental import pallas as pl
from jax.experimental.pallas import tpu as pltpu
from jax.experimental.pallas import tpu_sc as plsc

BN_EPS = 1e-5
_NW = 32
_PB = 1024


def _pair_body(in0_ref, in1_ref, out_ref):
    ey = (lax.broadcasted_iota(jnp.int32, (64, 64), 0)
          == lax.broadcasted_iota(jnp.int32, (64, 64), 1)).astype(jnp.float32)
    dn = (((0,), (0,)), ((), ()))
    out_ref[:, 0:64] = lax.dot_general(
        in0_ref[...], ey, dn, precision=lax.Precision.HIGHEST,
        preferred_element_type=jnp.float32)
    out_ref[:, 64:128] = lax.dot_general(
        in1_ref[...], ey, dn, precision=lax.Precision.HIGHEST,
        preferred_element_type=jnp.float32)


def _make_pair_table(V, D):
    npairs = (V + 2 * _PB - 1) // (2 * _PB)
    last_blk = (V + _PB - 1) // _PB - 1

    def run(tblT):
        return pl.pallas_call(
            _pair_body,
            grid=(npairs,),
            in_specs=[
                pl.BlockSpec((D, _PB), lambda j: (0, 2 * j)),
                pl.BlockSpec((D, _PB), lambda j: (0, jnp.minimum(2 * j + 1, last_blk))),
            ],
            out_specs=pl.BlockSpec((_PB, 2 * D), lambda j: (j, 0)),
            out_shape=jax.ShapeDtypeStruct((npairs * _PB, 2 * D), jnp.float32),
        )(tblT, tblT)

    return run


def _make_sc_gather(num_rows, W):
    b_per_w = num_rows // _NW
    K = b_per_w // 128
    mesh = plsc.VectorSubcoreMesh(core_axis_name="c", subcore_axis_name="s")

    @functools.partial(
        pl.kernel,
        mesh=mesh,
        compiler_params=pltpu.CompilerParams(use_tc_tiling_on_sc=False),
        out_type=jax.ShapeDtypeStruct((num_rows, W), jnp.float32),
        scratch_types=[
            pltpu.VMEM((K, 128), jnp.int32),
            pltpu.VMEM((128, W), jnp.float32),
            pltpu.VMEM((128, W), jnp.float32),
            pltpu.SemaphoreType.DMA,
            pltpu.SemaphoreType.DMA,
        ],
    )
    def gather_k(idx_hbm, tbl_hbm, out_hbm, idx_v, r0, r1, s0, s1):
        wid = lax.axis_index("s") * 2 + lax.axis_index("c")
        base = wid * b_per_w
        pltpu.sync_copy(idx_hbm.at[wid], idx_v)
        rs, ss = (r0, r1), (s0, s1)
        cps = [None, None]
        cps[0] = pltpu.async_copy(tbl_hbm.at[idx_v.at[0]], r0, s0)
        for j in range(K):
            if j + 1 < K:
                cps[(j + 1) % 2] = pltpu.async_copy(
                    tbl_hbm.at[idx_v.at[j + 1]], rs[(j + 1) % 2], ss[(j + 1) % 2])
            cps[j % 2].wait()
            pltpu.sync_copy(rs[j % 2], out_hbm.at[pl.ds(base + j * 128, 128)])

    return gather_k


def _half_body(g_ref, half_ref, out_ref):
    D = out_ref.shape[1]
    out_ref[...] = jnp.where(half_ref[...] != 0, g_ref[:, D:2 * D], g_ref[:, 0:D])


def _tc_body(rows_ref, rels_ref, gamma_ref, beta_ref, relw_t_ref,
             loss_ref, preds_ref):
    B = rels_ref.shape[0]
    R = preds_ref.shape[1]

    rows = rows_ref[...]

    def encode(x):
        n = jnp.sqrt(jnp.sum(x * x, axis=1, keepdims=True))
        x = jnp.where(n > 1.0, x / (n + 1e-7), x)
        mean = jnp.mean(x, axis=0, keepdims=True)
        var = jnp.mean((x - mean) ** 2, axis=0, keepdims=True)
        x = (x - mean) / jnp.sqrt(var + BN_EPS)
        return x * gamma_ref[...] + beta_ref[...]

    h = encode(rows[:B])
    t = encode(rows[B:])
    logits = jnp.dot(h * t, relw_t_ref[...], preferred_element_type=jnp.float32)
    m = jnp.max(logits, axis=1, keepdims=True)
    ex = jnp.exp(logits - m)
    s = jnp.sum(ex, axis=1, keepdims=True)
    lp = logits - m - jnp.log(s)
    preds_ref[...] = ex / s
    onehot = lax.broadcasted_iota(jnp.int32, (B, R), 1) == rels_ref[...]
    picked = jnp.sum(jnp.where(onehot, lp, 0.0), axis=0, keepdims=True)
    loss_ref[...] = -jnp.sum(picked, axis=1, keepdims=True) / B


def kernel(pos_edges, emb_table, bn_gamma, bn_beta, rel_w):
    B = pos_edges.shape[0]
    V, D = emb_table.shape
    R = rel_w.shape[0]

    num_rows = 2 * B
    b_per_w = num_rows // _NW

    pair_tbl = _make_pair_table(V, D)(emb_table.T)

    idx = jnp.concatenate([pos_edges[:, 0], pos_edges[:, 2]], axis=0)
    blk = idx >> 11
    rem = idx & (2 * _PB - 1)
    half = (rem >> 10) & 1
    pidx = (blk << 10) | (rem & (_PB - 1))
    idx3 = pidx.reshape(_NW, b_per_w // 128, 128)

    gathered = _make_sc_gather(num_rows, 2 * D)(idx3, pair_tbl)

    half2d = half.reshape(num_rows, 1)
    nhb = 8
    hb = num_rows // nhb
    rows = pl.pallas_call(
        _half_body,
        grid=(nhb,),
        in_specs=[
            pl.BlockSpec((hb, 2 * D), lambda j: (j, 0)),
            pl.BlockSpec((hb, 1), lambda j: (j, 0)),
        ],
        out_specs=pl.BlockSpec((hb, D), lambda j: (j, 0)),
        out_shape=jax.ShapeDtypeStruct((num_rows, D), jnp.float32),
    )(gathered, half2d)

    rels2d = pos_edges[:, 1].reshape(B, 1)
    gamma2d = bn_gamma.reshape(1, D)
    beta2d = bn_beta.reshape(1, D)
    relw_t = rel_w.T

    loss2d, preds = pl.pallas_call(
        _tc_body,
        out_shape=[
            jax.ShapeDtypeStruct((1, 1), jnp.float32),
            jax.ShapeDtypeStruct((B, R), jnp.float32),
        ],
    )(rows, rels2d, gamma2d, beta2d, relw_t)

    return (loss2d[0, 0], preds)


# speedup vs baseline: 1.9802x; 1.9802x over previous
import functools

import jax
import jax.numpy as jnp
from jax import lax
from jax.experimental import pallas as pl
from jax.experimental.pallas import tpu as pltpu
from jax.experimental.pallas import tpu_sc as plsc

BN_EPS = 1e-5
_NW = 32
_PB = 1024
_SB = 4096


def _pair_body(in0_ref, in1_ref, out_ref):
    ey = (lax.broadcasted_iota(jnp.int32, (64, 64), 0)
          == lax.broadcasted_iota(jnp.int32, (64, 64), 1)).astype(jnp.float32)
    dn = (((0,), (0,)), ((), ()))
    D = in0_ref.shape[0]
    nb = in0_ref.shape[1] // _PB
    for q in range(nb):
        sl = pl.ds(q * _PB, _PB)
        out_ref[q * _PB:(q + 1) * _PB, 0:D] = lax.dot_general(
            in0_ref[:, sl], ey, dn, preferred_element_type=jnp.float32)
        out_ref[q * _PB:(q + 1) * _PB, D:2 * D] = lax.dot_general(
            in1_ref[:, sl], ey, dn, preferred_element_type=jnp.float32)


def _make_pair_table(V, D):
    npairs = (V + 2 * _SB - 1) // (2 * _SB)
    last_blk = (V + _SB - 1) // _SB - 1

    def run(tblT):
        return pl.pallas_call(
            _pair_body,
            grid=(npairs,),
            in_specs=[
                pl.BlockSpec((D, _SB), lambda j: (0, 2 * j)),
                pl.BlockSpec((D, _SB), lambda j: (0, jnp.minimum(2 * j + 1, last_blk))),
            ],
            out_specs=pl.BlockSpec((_SB, 2 * D), lambda j: (j, 0)),
            out_shape=jax.ShapeDtypeStruct((npairs * _SB, 2 * D), jnp.float32),
        )(tblT, tblT)

    return run


def _make_sc_gather(num_rows, W):
    b_per_w = num_rows // _NW
    K = b_per_w // 128
    mesh = plsc.VectorSubcoreMesh(core_axis_name="c", subcore_axis_name="s")

    @functools.partial(
        pl.kernel,
        mesh=mesh,
        compiler_params=pltpu.CompilerParams(use_tc_tiling_on_sc=False),
        out_type=jax.ShapeDtypeStruct((num_rows, W), jnp.float32),
        scratch_types=[
            pltpu.VMEM((K, 128), jnp.int32),
            pltpu.VMEM((128, W), jnp.float32),
            pltpu.VMEM((128, W), jnp.float32),
            pltpu.SemaphoreType.DMA,
            pltpu.SemaphoreType.DMA,
        ],
    )
    def gather_k(idx_hbm, tbl_hbm, out_hbm, idx_v, r0, r1, s0, s1):
        wid = lax.axis_index("s") * 2 + lax.axis_index("c")
        base = wid * b_per_w
        pltpu.sync_copy(idx_hbm.at[wid], idx_v)
        rs, ss = (r0, r1), (s0, s1)
        cps = [None, None]
        cps[0] = pltpu.async_copy(tbl_hbm.at[idx_v.at[0]], r0, s0)
        for j in range(K):
            if j + 1 < K:
                cps[(j + 1) % 2] = pltpu.async_copy(
                    tbl_hbm.at[idx_v.at[j + 1]], rs[(j + 1) % 2], ss[(j + 1) % 2])
            cps[j % 2].wait()
            pltpu.sync_copy(rs[j % 2], out_hbm.at[pl.ds(base + j * 128, 128)])

    return gather_k


def _half_body(g_ref, half_ref, out_ref):
    D = out_ref.shape[1]
    out_ref[...] = jnp.where(half_ref[...] != 0, g_ref[:, D:2 * D], g_ref[:, 0:D])


def _tc_body(rows_ref, rels_ref, gamma_ref, beta_ref, relw_t_ref,
             loss_ref, preds_ref):
    B = rels_ref.shape[0]
    R = preds_ref.shape[1]

    rows = rows_ref[...]

    def encode(x):
        n = jnp.sqrt(jnp.sum(x * x, axis=1, keepdims=True))
        x = jnp.where(n > 1.0, x / (n + 1e-7), x)
        mean = jnp.mean(x, axis=0, keepdims=True)
        var = jnp.mean((x - mean) ** 2, axis=0, keepdims=True)
        x = (x - mean) / jnp.sqrt(var + BN_EPS)
        return x * gamma_ref[...] + beta_ref[...]

    h = encode(rows[:B])
    t = encode(rows[B:])
    logits = jnp.dot(h * t, relw_t_ref[...], preferred_element_type=jnp.float32)
    m = jnp.max(logits, axis=1, keepdims=True)
    ex = jnp.exp(logits - m)
    s = jnp.sum(ex, axis=1, keepdims=True)
    lp = logits - m - jnp.log(s)
    preds_ref[...] = ex / s
    onehot = lax.broadcasted_iota(jnp.int32, (B, R), 1) == rels_ref[...]
    picked = jnp.sum(jnp.where(onehot, lp, 0.0), axis=0, keepdims=True)
    loss_ref[...] = -jnp.sum(picked, axis=1, keepdims=True) / B


def kernel(pos_edges, emb_table, bn_gamma, bn_beta, rel_w):
    B = pos_edges.shape[0]
    V, D = emb_table.shape
    R = rel_w.shape[0]

    num_rows = 2 * B
    b_per_w = num_rows // _NW

    pair_tbl = _make_pair_table(V, D)(emb_table.T)

    idx = jnp.concatenate([pos_edges[:, 0], pos_edges[:, 2]], axis=0)
    blk = idx >> 13
    rem = idx & (2 * _SB - 1)
    half = (rem >> 12) & 1
    pidx = (blk << 12) | (rem & (_SB - 1))
    idx3 = pidx.reshape(_NW, b_per_w // 128, 128)

    gathered = _make_sc_gather(num_rows, 2 * D)(idx3, pair_tbl)

    half2d = half.reshape(num_rows, 1)
    nhb = 8
    hb = num_rows // nhb
    rows = pl.pallas_call(
        _half_body,
        grid=(nhb,),
        in_specs=[
            pl.BlockSpec((hb, 2 * D), lambda j: (j, 0)),
            pl.BlockSpec((hb, 1), lambda j: (j, 0)),
        ],
        out_specs=pl.BlockSpec((hb, D), lambda j: (j, 0)),
        out_shape=jax.ShapeDtypeStruct((num_rows, D), jnp.float32),
    )(gathered, half2d)

    rels2d = pos_edges[:, 1].reshape(B, 1)
    gamma2d = bn_gamma.reshape(1, D)
    beta2d = bn_beta.reshape(1, D)
    relw_t = rel_w.T

    loss2d, preds = pl.pallas_call(
        _tc_body,
        out_shape=[
            jax.ShapeDtypeStruct((1, 1), jnp.float32),
            jax.ShapeDtypeStruct((B, R), jnp.float32),
        ],
    )(rows, rels2d, gamma2d, beta2d, relw_t)

    return (loss2d[0, 0], preds)
